# baseline (speedup 1.0000x reference)
import functools

import jax
import jax.numpy as jnp
from jax import lax
from jax.experimental import pallas as pl
from jax.experimental.pallas import tpu as pltpu
from jax.experimental.pallas import tpu_sc as plsc

_SEQ = 8192
_DIM = 1024
_TC_ROWS = 5120
_SC_ROWS = _SEQ - _TC_ROWS

_info = plsc.get_sparse_core_info()
_NC, _NS = _info.num_cores, _info.num_subcores
_NW = _NC * _NS
_ROWS_PER_W = _SC_ROWS // _NW

_CH = 32
_NCHUNK = _ROWS_PER_W // _CH
_NB = 3

_BLK = 512

_mesh = plsc.VectorSubcoreMesh(core_axis_name="c", subcore_axis_name="s")


@functools.partial(
    pl.kernel,
    mesh=_mesh,
    out_type=jax.ShapeDtypeStruct((_SC_ROWS, _DIM), jnp.float32),
    scratch_types=(
        [pltpu.VMEM((_CH, _DIM), jnp.float32) for _ in range(_NB)]
        + [pltpu.SemaphoreType.DMA for _ in range(_NB)]
        + [pltpu.SemaphoreType.DMA for _ in range(_NB)]
    ),
)
def _sc_copy(table_hbm, out_hbm, *scratch):
    bufs = scratch[:_NB]
    rsems = scratch[_NB:2 * _NB]
    wsems = scratch[2 * _NB:]

    wid = lax.axis_index("s") * _NC + lax.axis_index("c")
    src_base = _TC_ROWS + wid * _ROWS_PER_W
    dst_base = wid * _ROWS_PER_W

    reads = [None] * _NCHUNK
    writes = [None] * _NCHUNK
    for i in range(min(_NB, _NCHUNK)):
        reads[i] = pltpu.async_copy(
            table_hbm.at[pl.ds(src_base + i * _CH, _CH)], bufs[i], rsems[i]
        )
    for i in range(_NCHUNK):
        b = i % _NB
        reads[i].wait()
        writes[i] = pltpu.async_copy(
            bufs[b], out_hbm.at[pl.ds(dst_base + i * _CH, _CH)], wsems[b]
        )
        j = i + _NB
        if j < _NCHUNK:
            writes[i].wait()
            reads[j] = pltpu.async_copy(
                table_hbm.at[pl.ds(src_base + j * _CH, _CH)], bufs[b], rsems[b]
            )
    for i in range(max(0, _NCHUNK - _NB), _NCHUNK):
        writes[i].wait()


def _tc_body(in_ref, out_ref):
    out_ref[...] = in_ref[...]


def kernel(hidden_embs, position_embedding_table):
    del hidden_embs
    tc_part = pl.pallas_call(
        _tc_body,
        grid=(_TC_ROWS // _BLK,),
        in_specs=[pl.BlockSpec((_BLK, _DIM), lambda i: (i, 0))],
        out_specs=pl.BlockSpec((_BLK, _DIM), lambda i: (i, 0)),
        out_shape=jax.ShapeDtypeStruct((_TC_ROWS, _DIM), jnp.float32),
    )(position_embedding_table)
    sc_part = _sc_copy(position_embedding_table)
    return jnp.concatenate([tc_part, sc_part], axis=0)
